# baseline (speedup 1.0000x reference)
import dataclasses
import functools

import jax
import jax.numpy as jnp
from jax import lax
from jax.experimental import pallas as pl
from jax.experimental.pallas import tpu as pltpu
from jax.experimental.pallas import tpu_sc as plsc

N_S, N_I, E, D, H, DE = 50000, 10000, 512000, 128, 128, 16
NSP = 50176
NIP = 10240
NCORES, NSUB = 2, 16
NW = NCORES * NSUB
EPW = E // NW
C = 128
SCHUNK = 3200
F32 = jnp.float32
HIGH = jax.lax.Precision.HIGHEST

_mesh = plsc.VectorSubcoreMesh(core_axis_name="c", subcore_axis_name="s")

_sc_params = pltpu.CompilerParams()
if "needs_layout_passes" in pltpu.CompilerParams.__dataclass_fields__:
    _sc_params = dataclasses.replace(_sc_params, needs_layout_passes=False)


BIGN = 16640
ZR = 32
TSH = 13


def _zero_acc_share(zbuf, acc, sub, rows, sem):
    zrows = rows // NSUB
    row0 = sub * zrows
    spans = []
    done = 0
    while done < zrows:
        sz = min(ZR, zrows - done)
        spans.append((done, sz))
        done += sz
    for d, sz in spans:
        pltpu.async_copy(zbuf.at[pl.ds(0, sz)],
                         acc.at[pl.ds(row0 + d, sz)], sem)
    for d, sz in spans:
        pltpu.make_async_copy(zbuf.at[pl.ds(0, sz)],
                              acc.at[pl.ds(row0 + d, sz)], sem).wait()


def _run_pipe(table, acc, big, nch, rowss, gcurs, tcurs, semgs):
    def unpack(k, slot):
        gc, tc = gcurs[slot], tcurs[slot]
        for j in range(C // 16):
            v = big[pl.ds(k * C + j * 16, 16)]
            gc[pl.ds(j * 16, 16)] = jax.lax.shift_right_logical(v, TSH)
            tc[pl.ds(j * 16, 16)] = jax.lax.bitwise_and(v, (1 << TSH) - 1)

    for slot in range(2):
        unpack(slot, slot)
        pltpu.async_copy(table.at[gcurs[slot].at[pl.ds(0, C)]],
                         rowss[slot], semgs[slot])

    def unpack_dyn(k, slot):
        gc, tc = gcurs[slot], tcurs[slot]
        for j in range(C // 16):
            v = big[pl.ds(k * C + j * 16, 16)]
            gc[pl.ds(j * 16, 16)] = jax.lax.shift_right_logical(v, TSH)
            tc[pl.ds(j * 16, 16)] = jax.lax.bitwise_and(v, (1 << TSH) - 1)

    def quad(q):
        for u in range(4):
            k = 4 * q + u
            s = u % 2
            rb, sg = rowss[s], semgs[s]
            pltpu.make_async_copy(table.at[gcurs[s].at[pl.ds(0, C)]],
                                  rb, sg).wait()
            pltpu.sync_copy(rb, acc.at[tcurs[s]], add=True)

            @pl.when(k + 2 < nch)
            def _():
                unpack_dyn(k + 2, s)
                pltpu.async_copy(table.at[gcurs[s].at[pl.ds(0, C)]], rb, sg)

    pl.loop(0, nch // 4)(quad)


def _agg_body(nout, rs, *refs):
    nranges = nout // rs
    (tableA, tableB, gidx, sidx, out_pA, out_pB, acc, gb0, sb0, gb1, sb1,
     big, gcur0, gcur1, tcur0, tcur1, rows0, rows1, zbuf,
     semg0, semg1, seme0, seme1, semz) = refs
    gbufs, sbufs, semes = (gb0, gb1), (sb0, sb1), (seme0, seme1)
    gcurs, tcurs = (gcur0, gcur1), (tcur0, tcur1)
    rowss, semgs = (rows0, rows1), (semg0, semg1)

    core = lax.axis_index("c")
    sub = lax.axis_index("s")
    ebase = (core * NSUB + sub) * EPW
    rs_per = rs // NSUB
    NSC = EPW // SCHUNK

    zeros16 = jnp.zeros((16,), F32)

    @pl.loop(0, ZR)
    def _(rr):
        for j in range(H // 16):
            zbuf[rr, pl.ds(j * 16, 16)] = zeros16

    trash16 = jnp.full((16,), rs, jnp.int32)

    @pl.loop(0, nranges)
    def _range_body(r):
        base = r * rs
        _zero_issue = [pltpu.async_copy(
            zbuf.at[pl.ds(0, min(ZR, rs // NSUB - d))],
            acc.at[pl.ds(sub * rs_per + d, min(ZR, rs // NSUB - d))], semz)
            for d in range(0, rs // NSUB, ZR)]

        pltpu.async_copy(gidx.at[pl.ds(ebase, SCHUNK)], gbufs[0], semes[0])
        pltpu.async_copy(sidx.at[pl.ds(ebase, SCHUNK)], sbufs[0], semes[0])
        n = jnp.zeros((16,), jnp.int32)
        for sc in range(NSC):
            cur = sc % 2
            pltpu.make_async_copy(gidx.at[pl.ds(ebase, SCHUNK)],
                                  gbufs[cur], semes[cur]).wait()
            pltpu.make_async_copy(sidx.at[pl.ds(ebase, SCHUNK)],
                                  sbufs[cur], semes[cur]).wait()
            if sc + 1 < NSC:
                off = ebase + (sc + 1) * SCHUNK
                pltpu.async_copy(gidx.at[pl.ds(off, SCHUNK)],
                                 gbufs[cur ^ 1], semes[cur ^ 1])
                pltpu.async_copy(sidx.at[pl.ds(off, SCHUNK)],
                                 sbufs[cur ^ 1], semes[cur ^ 1])

            def comp(q, nv, _gb=gbufs[cur], _sb=sbufs[cur]):
                for jj in range(4):
                    off = q * 64 + jj * 16
                    g16 = _gb[pl.ds(off, 16)]
                    s16 = _sb[pl.ds(off, 16)]
                    t = s16 - base
                    m = jnp.logical_and(t >= 0, t < rs)
                    mi = m.astype(jnp.int32)
                    pos = nv + plsc.cumsum(mi) - 1
                    packed = jax.lax.bitwise_or(
                        jax.lax.shift_left(g16, TSH), t)
                    plsc.store_scatter(big, [pos], packed, mask=m)
                    nv = nv + plsc.all_reduce_population_count(m)
                return nv

            n = lax.fori_loop(0, SCHUNK // 64, comp, n)

        n = jnp.sum(n) >> 4
        for j in range(4 * C // 16):
            big[pl.ds(n + j * 16, 16)] = trash16
        nch = jnp.maximum((n + C - 1) // C, 4)
        nch = (nch + 3) // 4 * 4

        for d in range(0, rs // NSUB, ZR):
            sz = min(ZR, rs // NSUB - d)
            pltpu.make_async_copy(zbuf.at[pl.ds(0, sz)],
                                  acc.at[pl.ds(sub * rs_per + d, sz)],
                                  semz).wait()
        plsc.subcore_barrier()

        _run_pipe(tableA, acc, big, nch, rowss, gcurs, tcurs, semgs)
        plsc.subcore_barrier()
        pltpu.sync_copy(acc.at[pl.ds(sub * rs_per, rs_per)],
                        out_pA.at[core, pl.ds(base + sub * rs_per, rs_per)])
        _zero_acc_share(zbuf, acc, sub, rs, semz)
        plsc.subcore_barrier()
        _run_pipe(tableB, acc, big, nch, rowss, gcurs, tcurs, semgs)
        plsc.subcore_barrier()
        pltpu.sync_copy(acc.at[pl.ds(sub * rs_per, rs_per)],
                        out_pB.at[core, pl.ds(base + sub * rs_per, rs_per)])


def _aggregate2(tableA, tableB, gidx, sidx, nout, rs):
    acc_rows = rs + 16
    scratch = [
        pltpu.VMEM_SHARED((acc_rows, H), F32),
        pltpu.VMEM((SCHUNK,), jnp.int32),
        pltpu.VMEM((SCHUNK,), jnp.int32),
        pltpu.VMEM((SCHUNK,), jnp.int32),
        pltpu.VMEM((SCHUNK,), jnp.int32),
        pltpu.VMEM((BIGN,), jnp.int32),
        pltpu.VMEM((C,), jnp.int32),
        pltpu.VMEM((C,), jnp.int32),
        pltpu.VMEM((C,), jnp.int32),
        pltpu.VMEM((C,), jnp.int32),
        pltpu.VMEM((C, H), F32),
        pltpu.VMEM((C, H), F32),
        pltpu.VMEM((ZR, H), F32),
        pltpu.SemaphoreType.DMA,
        pltpu.SemaphoreType.DMA,
        pltpu.SemaphoreType.DMA,
        pltpu.SemaphoreType.DMA,
        pltpu.SemaphoreType.DMA,
    ]
    fn = pl.kernel(
        functools.partial(_agg_body, nout, rs),
        out_type=(jax.ShapeDtypeStruct((NCORES, nout, H), F32),
                  jax.ShapeDtypeStruct((NCORES, nout, H), F32)),
        mesh=_mesh,
        scratch_types=scratch,
        compiler_params=_sc_params,
    )
    return fn(tableA, tableB, gidx, sidx)


def _zero_acc_once(rows0, acc, sub, rows):
    zeros16 = jnp.zeros((16,), F32)

    @pl.loop(0, C)
    def _(rr):
        for j in range(H // 16):
            rows0[rr, pl.ds(j * 16, 16)] = zeros16

    zrows = rows // NSUB
    row0 = sub * zrows
    done = 0
    while done < zrows:
        sz = min(C, zrows - done)
        pltpu.sync_copy(rows0.at[pl.ds(0, sz)],
                        acc.at[pl.ds(row0 + done, sz)])
        done += sz


def _agg_items_body(*refs):
    (table, gidx, sidx, out_p, acc, gb0, sb0, gb1, sb1,
     tcur0, tcur1, rows0, rows1, semg0, semg1, seme0, seme1) = refs
    gbufs, sbufs, semes = (gb0, gb1), (sb0, sb1), (seme0, seme1)
    tcurs, rowss, semgs = (tcur0, tcur1), (rows0, rows1), (semg0, semg1)

    core = lax.axis_index("c")
    sub = lax.axis_index("s")
    ebase = (core * NSUB + sub) * EPW
    acc_rows = NIP + 16
    NSC = EPW // SCHUNK
    NCH_S = SCHUNK // C
    NCH = NSC * NCH_S

    _zero_acc_once(rows0, acc, sub, acc_rows)
    plsc.subcore_barrier()

    def _wait_ebuf(slot, off):
        pltpu.make_async_copy(gidx.at[pl.ds(off, SCHUNK)],
                              gbufs[slot], semes[slot]).wait()
        pltpu.make_async_copy(sidx.at[pl.ds(off, SCHUNK)],
                              sbufs[slot], semes[slot]).wait()

    def _load_ebuf(slot, off):
        pltpu.async_copy(gidx.at[pl.ds(off, SCHUNK)], gbufs[slot],
                         semes[slot])
        pltpu.async_copy(sidx.at[pl.ds(off, SCHUNK)], sbufs[slot],
                         semes[slot])

    _load_ebuf(0, ebase)
    _wait_ebuf(0, ebase)
    pltpu.async_copy(table.at[gbufs[0].at[pl.ds(0, C)]], rows0, semg0)
    pltpu.async_copy(table.at[gbufs[0].at[pl.ds(C, C)]], rows1, semg1)

    for sc in range(NSC):
        cur = sc % 2
        if sc + 1 < NSC:
            _load_ebuf(cur ^ 1, ebase + (sc + 1) * SCHUNK)
        for cl in range(NCH_S):
            k = sc * NCH_S + cl
            par = k % 2
            rb, tc, sg = rowss[par], tcurs[par], semgs[par]
            if cl == NCH_S - 2 and sc + 1 < NSC:
                _wait_ebuf(cur ^ 1, ebase + (sc + 1) * SCHUNK)
            pltpu.make_async_copy(table.at[gbufs[0].at[pl.ds(0, C)]],
                                  rb, sg).wait()
            sb_ = sbufs[cur]
            for j in range(C // 16):
                tc[pl.ds(j * 16, 16)] = sb_[pl.ds(cl * C + j * 16, 16)]
            pltpu.sync_copy(rb, acc.at[tc], add=True)
            k2 = k + 2
            if k2 < NCH:
                sc2, cl2 = divmod(k2, NCH_S)
                pltpu.async_copy(
                    table.at[gbufs[sc2 % 2].at[pl.ds(cl2 * C, C)]], rb, sg)

    plsc.subcore_barrier()
    rs_per = NIP // NSUB
    pltpu.sync_copy(acc.at[pl.ds(sub * rs_per, rs_per)],
                    out_p.at[core, pl.ds(sub * rs_per, rs_per)])


def _aggregate_items(table, gidx, sidx):
    scratch = [
        pltpu.VMEM_SHARED((NIP + 16, H), F32),
        pltpu.VMEM((SCHUNK,), jnp.int32),
        pltpu.VMEM((SCHUNK,), jnp.int32),
        pltpu.VMEM((SCHUNK,), jnp.int32),
        pltpu.VMEM((SCHUNK,), jnp.int32),
        pltpu.VMEM((C,), jnp.int32),
        pltpu.VMEM((C,), jnp.int32),
        pltpu.VMEM((C, H), F32),
        pltpu.VMEM((C, H), F32),
        pltpu.SemaphoreType.DMA,
        pltpu.SemaphoreType.DMA,
        pltpu.SemaphoreType.DMA,
        pltpu.SemaphoreType.DMA,
    ]
    fn = pl.kernel(
        _agg_items_body,
        out_type=jax.ShapeDtypeStruct((NCORES, NIP, H), F32),
        mesh=_mesh,
        scratch_types=scratch,
        compiler_params=_sc_params,
    )
    return fn(table, gidx, sidx)


def _count_body(src, dst, out_cs, out_ci, cs, ci, gb, sb, sem):
    core = lax.axis_index("c")
    sub = lax.axis_index("s")
    ebase = (core * NSUB + sub) * EPW
    zeros16 = jnp.zeros((16,), F32)
    ones16 = jnp.full((16,), 1.0, F32)

    @pl.loop(0, NSP // 16)
    def _(q):
        cs[pl.ds(q * 16, 16)] = zeros16

    @pl.loop(0, NIP // 16)
    def _(q):
        ci[pl.ds(q * 16, 16)] = zeros16

    for sc in range(EPW // SCHUNK):
        pltpu.sync_copy(src.at[pl.ds(ebase + sc * SCHUNK, SCHUNK)], sb)
        pltpu.sync_copy(dst.at[pl.ds(ebase + sc * SCHUNK, SCHUNK)], gb)

        @pl.loop(0, SCHUNK // 64)
        def _(q):
            for jj in range(4):
                off = q * 64 + jj * 16
                s16 = sb[pl.ds(off, 16)]
                d16 = gb[pl.ds(off, 16)]
                plsc.addupdate_scatter(cs, [s16], ones16)
                plsc.addupdate_scatter(ci, [d16], ones16)

    pltpu.sync_copy(cs, out_cs.at[core, sub])
    pltpu.sync_copy(ci, out_ci.at[core, sub])


def _counts(edge_src, edge_dst):
    fn = pl.kernel(
        _count_body,
        out_type=(jax.ShapeDtypeStruct((NCORES, NSUB, NSP), F32),
                  jax.ShapeDtypeStruct((NCORES, NSUB, NIP), F32)),
        mesh=_mesh,
        scratch_types=[
            pltpu.VMEM((NSP,), F32),
            pltpu.VMEM((NIP,), F32),
            pltpu.VMEM((SCHUNK,), jnp.int32),
            pltpu.VMEM((SCHUNK,), jnp.int32),
            pltpu.SemaphoreType.DMA,
        ],
        compiler_params=_sc_params,
    )
    return fn(edge_src, edge_dst)


FC = 2000


def _final_body(a_hbm, b_hbm, c_hbm, src_hbm, dst_hbm, out_hbm,
                a_v, b_v, sbuf, dbuf, cbuf, obuf, sem):
    core = lax.axis_index("c")
    sub = lax.axis_index("s")
    wid = core * NSUB + sub
    base = wid * EPW
    pltpu.sync_copy(a_hbm, a_v)
    pltpu.sync_copy(b_hbm, b_v)
    for ch in range(EPW // FC):
        off = base + ch * FC
        pltpu.sync_copy(src_hbm.at[pl.ds(off, FC)], sbuf)
        pltpu.sync_copy(dst_hbm.at[pl.ds(off, FC)], dbuf)
        pltpu.sync_copy(c_hbm.at[pl.ds(off, FC)], cbuf)

        @pl.loop(0, FC // 16)
        def _(j):
            s16 = sbuf[pl.ds(j * 16, 16)]
            d16 = dbuf[pl.ds(j * 16, 16)]
            av = plsc.load_gather(a_v, [s16])
            bv = plsc.load_gather(b_v, [d16])
            cv = cbuf[pl.ds(j * 16, 16)]
            obuf[pl.ds(j * 16, 16)] = av + bv + cv

        pltpu.sync_copy(obuf, out_hbm.at[pl.ds(off, FC)])


def _final(a_s, b_i, c_e, src, dst):
    fn = pl.kernel(
        _final_body,
        out_type=jax.ShapeDtypeStruct((E,), F32),
        mesh=_mesh,
        scratch_types=[
            pltpu.VMEM((NSP,), F32),
            pltpu.VMEM((NIP,), F32),
            pltpu.VMEM((FC,), jnp.int32),
            pltpu.VMEM((FC,), jnp.int32),
            pltpu.VMEM((FC,), F32),
            pltpu.VMEM((FC,), F32),
            pltpu.SemaphoreType.DMA,
        ],
        compiler_params=_sc_params,
    )
    return fn(a_s, b_i, c_e, src, dst)


BR = 512


def _embed_tc_body(x_ref, w_ref, b_ref, e_ref, o_ref):
    o_ref[...] = (jnp.dot(x_ref[...], w_ref[...], preferred_element_type=F32,
                          precision=HIGH) + b_ref[...] + e_ref[...])


def _embed_tc(x, w, b, emb):
    n = x.shape[0]
    return pl.pallas_call(
        _embed_tc_body,
        grid=(n // BR,),
        in_specs=[pl.BlockSpec((BR, D), lambda i: (i, 0)),
                  pl.BlockSpec((D, H), lambda i: (0, 0)),
                  pl.BlockSpec((1, H), lambda i: (0, 0)),
                  pl.BlockSpec((BR, H), lambda i: (i, 0))],
        out_specs=pl.BlockSpec((BR, H), lambda i: (i, 0)),
        out_shape=jax.ShapeDtypeStruct((n, H), F32),
    )(x, w, b.reshape(1, H), emb)


def _sage_tc_body(relu, proj, xd_ref, p0_ref, p1_ref, cnt_ref, wr_ref, wn_ref,
                  b_ref, *rest):
    if proj:
        wp_ref, cb_ref, o_ref = rest
    else:
        (o_ref,) = rest
    cnt = jnp.sum(cnt_ref[...], axis=0)
    recip = 1.0 / jnp.maximum(cnt, 1.0)
    mean = (p0_ref[0] + p1_ref[0]) * recip[:, None]
    res = (jnp.dot(xd_ref[...], wr_ref[...], preferred_element_type=F32,
                   precision=HIGH)
           + jnp.dot(mean, wn_ref[...], preferred_element_type=F32,
                     precision=HIGH)
           + b_ref[...])
    if relu:
        res = jnp.maximum(res, 0.0)
    if proj:
        res = (jnp.dot(res, wp_ref[...], preferred_element_type=F32,
                       precision=HIGH) + cb_ref[...])
    o_ref[...] = res


def _sage_tc(xd, parts, colblk, cnt, wr, wn, b, relu=False, proj=None,
             proj_b=None):
    n = xd.shape[0]
    ins = [xd, parts, parts, cnt, wr, wn, b.reshape(1, H)]
    in_specs = [pl.BlockSpec((BR, H), lambda i: (i, 0)),
                pl.BlockSpec((1, BR, H), lambda i, cb=colblk: (0, i, cb)),
                pl.BlockSpec((1, BR, H), lambda i, cb=colblk: (1, i, cb)),
                pl.BlockSpec((NW, BR), lambda i: (0, i)),
                pl.BlockSpec((H, H), lambda i: (0, 0)),
                pl.BlockSpec((H, H), lambda i: (0, 0)),
                pl.BlockSpec((1, H), lambda i: (0, 0))]
    if proj is not None:
        ins += [proj, proj_b]
        in_specs += [pl.BlockSpec((H, H), lambda i: (0, 0)),
                     pl.BlockSpec((1, H), lambda i: (0, 0))]
    return pl.pallas_call(
        functools.partial(_sage_tc_body, relu, proj is not None),
        grid=(n // BR,),
        in_specs=in_specs,
        out_specs=pl.BlockSpec((BR, H), lambda i: (i, 0)),
        out_shape=jax.ShapeDtypeStruct((n, H), F32),
    )(*ins)


def _edgec_tc_body(ea_ref, w_ref, o_ref):
    o_ref[...] = jnp.dot(ea_ref[...], w_ref[...], preferred_element_type=F32,
                         precision=HIGH)


def _edgec_tc(ea, w):
    blk = 4096
    return pl.pallas_call(
        _edgec_tc_body,
        grid=(E // blk,),
        in_specs=[pl.BlockSpec((blk, DE), lambda i: (i, 0)),
                  pl.BlockSpec((DE, 8), lambda i: (0, 0))],
        out_specs=pl.BlockSpec((blk, 8), lambda i: (i, 0)),
        out_shape=jax.ShapeDtypeStruct((E, 8), F32),
    )(ea, w)


def kernel(student_x, item_x, student_node_id, item_node_id, edge_src,
           edge_dst, edge_attr, slW, slb, ilW, ilb, s_emb, i_emb,
           c1_si_r, c1_si_n, c1_si_b, c1_is_r, c1_is_n, c1_is_b,
           c2_si_r, c2_si_n, c2_si_b, c2_is_r, c2_is_n, c2_is_b,
           clsW, clsb):
    sx = jnp.pad(student_x, ((0, NSP - N_S), (0, 0)))
    se = jnp.pad(s_emb, ((0, NSP - N_S), (0, 0)))
    ix = jnp.pad(item_x, ((0, NIP - N_I), (0, 0)))
    ie = jnp.pad(i_emb, ((0, NIP - N_I), (0, 0)))

    s = _embed_tc(sx, slW, slb, se)
    i = _embed_tc(ix, ilW, ilb, ie)

    cs, ci = _counts(edge_src, edge_dst)
    cnt_i = ci.reshape(NW, NIP)
    cnt_s = cs.reshape(NW, NSP)
    pi = _aggregate_items(s, edge_src, edge_dst)
    i1 = _sage_tc(i, pi, 0, cnt_i, c1_si_r, c1_si_n, c1_si_b, relu=True)

    ps, ps2 = _aggregate2(i, i1, edge_dst, edge_src, NSP, NSP // 8)
    s1 = _sage_tc(s, ps, 0, cnt_s, c1_is_r, c1_is_n, c1_is_b, relu=True)
    pi2 = _aggregate_items(s1, edge_src, edge_dst)

    w_s = jnp.pad(clsW[:H], ((0, 0), (0, H - 1)))
    w_i = jnp.pad(clsW[H:2 * H], ((0, 0), (0, H - 1)))
    cb128 = jnp.pad(clsb.reshape(1, 1), ((0, 0), (0, H - 1)))
    zb128 = jnp.zeros((1, H), F32)

    a_s = _sage_tc(s1, ps2, 0, cnt_s, c2_is_r, c2_is_n, c2_is_b,
                   proj=w_s, proj_b=cb128)[:, 0]
    b_i = _sage_tc(i1, pi2, 0, cnt_i, c2_si_r, c2_si_n, c2_si_b,
                   proj=w_i, proj_b=zb128)[:, 0]

    w_e = jnp.pad(clsW[2 * H:], ((0, 0), (0, 7)))
    c_e = _edgec_tc(edge_attr, w_e)[:, 0]

    return _final(a_s, b_i, c_e, edge_src, edge_dst)


# speedup vs baseline: 1.2160x; 1.2160x over previous
import dataclasses
import functools

import jax
import jax.numpy as jnp
from jax import lax
from jax.experimental import pallas as pl
from jax.experimental.pallas import tpu as pltpu
from jax.experimental.pallas import tpu_sc as plsc

N_S, N_I, E, D, H, DE = 50000, 10000, 512000, 128, 128, 16
NSP = 50176
NIP = 10240
NCORES, NSUB = 2, 16
NW = NCORES * NSUB
EPW = E // NW
C = 128
SCHUNK = 3200
F32 = jnp.float32
HIGH = jax.lax.Precision.HIGHEST

_mesh = plsc.VectorSubcoreMesh(core_axis_name="c", subcore_axis_name="s")

_sc_params = pltpu.CompilerParams()
if "needs_layout_passes" in pltpu.CompilerParams.__dataclass_fields__:
    _sc_params = dataclasses.replace(_sc_params, needs_layout_passes=False)


BIGN = 16640
ZR = 32
TSH = 13


def _zero_acc_share(zbuf, acc, sub, rows, sem):
    zrows = rows // NSUB
    row0 = sub * zrows
    spans = []
    done = 0
    while done < zrows:
        sz = min(ZR, zrows - done)
        spans.append((done, sz))
        done += sz
    for d, sz in spans:
        pltpu.async_copy(zbuf.at[pl.ds(0, sz)],
                         acc.at[pl.ds(row0 + d, sz)], sem)
    for d, sz in spans:
        pltpu.make_async_copy(zbuf.at[pl.ds(0, sz)],
                              acc.at[pl.ds(row0 + d, sz)], sem).wait()


def _run_pipe(table, acc, big, nch, rowss, gcurs, tcurs, semgs):
    def unpack(k, slot):
        gc, tc = gcurs[slot], tcurs[slot]
        for j in range(C // 16):
            v = big[pl.ds(k * C + j * 16, 16)]
            gc[pl.ds(j * 16, 16)] = jax.lax.shift_right_logical(v, TSH)
            tc[pl.ds(j * 16, 16)] = jax.lax.bitwise_and(v, (1 << TSH) - 1)

    for slot in range(2):
        unpack(slot, slot)
        pltpu.async_copy(table.at[gcurs[slot].at[pl.ds(0, C)]],
                         rowss[slot], semgs[slot])

    def unpack_dyn(k, slot):
        gc, tc = gcurs[slot], tcurs[slot]
        for j in range(C // 16):
            v = big[pl.ds(k * C + j * 16, 16)]
            gc[pl.ds(j * 16, 16)] = jax.lax.shift_right_logical(v, TSH)
            tc[pl.ds(j * 16, 16)] = jax.lax.bitwise_and(v, (1 << TSH) - 1)

    def pair(q):
        for u in range(2):
            k = 2 * q + u
            rb, sg = rowss[u], semgs[u]
            pltpu.make_async_copy(table.at[gcurs[u].at[pl.ds(0, C)]],
                                  rb, sg).wait()
            pltpu.sync_copy(rb, acc.at[tcurs[u]], add=True)

            @pl.when(k + 2 < nch)
            def _():
                unpack_dyn(k + 2, u)
                pltpu.async_copy(table.at[gcurs[u].at[pl.ds(0, C)]], rb, sg)

    pl.loop(0, nch // 2)(pair)


def _agg_body(nout, rs, *refs):
    nranges = nout // rs
    (tableA, tableB, gidx, sidx, out_pA, out_pB, acc, gb0, sb0, gb1, sb1,
     big, gcur0, gcur1, tcur0, tcur1, rows0, rows1, zbuf,
     semg0, semg1, seme0, seme1, semz) = refs
    gbufs, sbufs, semes = (gb0, gb1), (sb0, sb1), (seme0, seme1)
    gcurs, tcurs = (gcur0, gcur1), (tcur0, tcur1)
    rowss, semgs = (rows0, rows1), (semg0, semg1)

    core = lax.axis_index("c")
    sub = lax.axis_index("s")
    ebase = (core * NSUB + sub) * EPW
    rs_per = rs // NSUB
    NSC = EPW // SCHUNK

    zeros16 = jnp.zeros((16,), F32)

    @pl.loop(0, ZR)
    def _(rr):
        for j in range(H // 16):
            zbuf[rr, pl.ds(j * 16, 16)] = zeros16

    trash16 = jnp.full((16,), rs, jnp.int32)

    for r in range(nranges):
        base = r * rs
        _zero_issue = [pltpu.async_copy(
            zbuf.at[pl.ds(0, min(ZR, rs // NSUB - d))],
            acc.at[pl.ds(sub * rs_per + d, min(ZR, rs // NSUB - d))], semz)
            for d in range(0, rs // NSUB, ZR)]

        pltpu.async_copy(gidx.at[pl.ds(ebase, SCHUNK)], gbufs[0], semes[0])
        pltpu.async_copy(sidx.at[pl.ds(ebase, SCHUNK)], sbufs[0], semes[0])
        n = jnp.zeros((16,), jnp.int32)
        for sc in range(NSC):
            cur = sc % 2
            pltpu.make_async_copy(gidx.at[pl.ds(ebase, SCHUNK)],
                                  gbufs[cur], semes[cur]).wait()
            pltpu.make_async_copy(sidx.at[pl.ds(ebase, SCHUNK)],
                                  sbufs[cur], semes[cur]).wait()
            if sc + 1 < NSC:
                off = ebase + (sc + 1) * SCHUNK
                pltpu.async_copy(gidx.at[pl.ds(off, SCHUNK)],
                                 gbufs[cur ^ 1], semes[cur ^ 1])
                pltpu.async_copy(sidx.at[pl.ds(off, SCHUNK)],
                                 sbufs[cur ^ 1], semes[cur ^ 1])

            def comp(q, nv, _gb=gbufs[cur], _sb=sbufs[cur]):
                for jj in range(4):
                    off = q * 64 + jj * 16
                    g16 = _gb[pl.ds(off, 16)]
                    s16 = _sb[pl.ds(off, 16)]
                    t = s16 - base
                    m = jnp.logical_and(t >= 0, t < rs)
                    mi = m.astype(jnp.int32)
                    pos = nv + plsc.cumsum(mi) - 1
                    packed = jax.lax.bitwise_or(
                        jax.lax.shift_left(g16, TSH), t)
                    plsc.store_scatter(big, [pos], packed, mask=m)
                    nv = nv + plsc.all_reduce_population_count(m)
                return nv

            n = lax.fori_loop(0, SCHUNK // 64, comp, n)

        n = jnp.sum(n) >> 4
        for j in range(2 * C // 16):
            big[pl.ds(n + j * 16, 16)] = trash16
        nch = jnp.maximum((n + C - 1) // C, 2)
        nch = (nch + 1) // 2 * 2

        for d in range(0, rs // NSUB, ZR):
            sz = min(ZR, rs // NSUB - d)
            pltpu.make_async_copy(zbuf.at[pl.ds(0, sz)],
                                  acc.at[pl.ds(sub * rs_per + d, sz)],
                                  semz).wait()
        plsc.subcore_barrier()

        _run_pipe(tableA, acc, big, nch, rowss, gcurs, tcurs, semgs)
        plsc.subcore_barrier()
        pltpu.sync_copy(acc.at[pl.ds(sub * rs_per, rs_per)],
                        out_pA.at[core, pl.ds(base + sub * rs_per, rs_per)])
        _zero_acc_share(zbuf, acc, sub, rs, semz)
        plsc.subcore_barrier()
        _run_pipe(tableB, acc, big, nch, rowss, gcurs, tcurs, semgs)
        plsc.subcore_barrier()
        pltpu.sync_copy(acc.at[pl.ds(sub * rs_per, rs_per)],
                        out_pB.at[core, pl.ds(base + sub * rs_per, rs_per)])


def _aggregate2(tableA, tableB, gidx, sidx, nout, rs):
    acc_rows = rs + 16
    scratch = [
        pltpu.VMEM_SHARED((acc_rows, H), F32),
        pltpu.VMEM((SCHUNK,), jnp.int32),
        pltpu.VMEM((SCHUNK,), jnp.int32),
        pltpu.VMEM((SCHUNK,), jnp.int32),
        pltpu.VMEM((SCHUNK,), jnp.int32),
        pltpu.VMEM((BIGN,), jnp.int32),
        pltpu.VMEM((C,), jnp.int32),
        pltpu.VMEM((C,), jnp.int32),
        pltpu.VMEM((C,), jnp.int32),
        pltpu.VMEM((C,), jnp.int32),
        pltpu.VMEM((C, H), F32),
        pltpu.VMEM((C, H), F32),
        pltpu.VMEM((ZR, H), F32),
        pltpu.SemaphoreType.DMA,
        pltpu.SemaphoreType.DMA,
        pltpu.SemaphoreType.DMA,
        pltpu.SemaphoreType.DMA,
        pltpu.SemaphoreType.DMA,
    ]
    fn = pl.kernel(
        functools.partial(_agg_body, nout, rs),
        out_type=(jax.ShapeDtypeStruct((NCORES, nout, H), F32),
                  jax.ShapeDtypeStruct((NCORES, nout, H), F32)),
        mesh=_mesh,
        scratch_types=scratch,
        compiler_params=_sc_params,
    )
    return fn(tableA, tableB, gidx, sidx)


def _zero_acc_once(rows0, acc, sub, rows):
    zeros16 = jnp.zeros((16,), F32)

    @pl.loop(0, C)
    def _(rr):
        for j in range(H // 16):
            rows0[rr, pl.ds(j * 16, 16)] = zeros16

    zrows = rows // NSUB
    row0 = sub * zrows
    done = 0
    while done < zrows:
        sz = min(C, zrows - done)
        pltpu.sync_copy(rows0.at[pl.ds(0, sz)],
                        acc.at[pl.ds(row0 + done, sz)])
        done += sz


def _agg_items_body(*refs):
    (table, gidx, sidx, out_p, acc, gb0, sb0, gb1, sb1,
     tcur0, tcur1, rows0, rows1, semg0, semg1, seme0, seme1) = refs
    gbufs, sbufs, semes = (gb0, gb1), (sb0, sb1), (seme0, seme1)
    tcurs, rowss, semgs = (tcur0, tcur1), (rows0, rows1), (semg0, semg1)

    core = lax.axis_index("c")
    sub = lax.axis_index("s")
    ebase = (core * NSUB + sub) * EPW
    acc_rows = NIP + 16
    NSC = EPW // SCHUNK
    NCH_S = SCHUNK // C
    NCH = NSC * NCH_S

    _zero_acc_once(rows0, acc, sub, acc_rows)
    plsc.subcore_barrier()

    def _wait_ebuf(slot, off):
        pltpu.make_async_copy(gidx.at[pl.ds(off, SCHUNK)],
                              gbufs[slot], semes[slot]).wait()
        pltpu.make_async_copy(sidx.at[pl.ds(off, SCHUNK)],
                              sbufs[slot], semes[slot]).wait()

    def _load_ebuf(slot, off):
        pltpu.async_copy(gidx.at[pl.ds(off, SCHUNK)], gbufs[slot],
                         semes[slot])
        pltpu.async_copy(sidx.at[pl.ds(off, SCHUNK)], sbufs[slot],
                         semes[slot])

    _load_ebuf(0, ebase)
    _wait_ebuf(0, ebase)
    pltpu.async_copy(table.at[gbufs[0].at[pl.ds(0, C)]], rows0, semg0)
    pltpu.async_copy(table.at[gbufs[0].at[pl.ds(C, C)]], rows1, semg1)

    for sc in range(NSC):
        cur = sc % 2
        if sc + 1 < NSC:
            _load_ebuf(cur ^ 1, ebase + (sc + 1) * SCHUNK)
        for cl in range(NCH_S):
            k = sc * NCH_S + cl
            par = k % 2
            rb, tc, sg = rowss[par], tcurs[par], semgs[par]
            if cl == NCH_S - 2 and sc + 1 < NSC:
                _wait_ebuf(cur ^ 1, ebase + (sc + 1) * SCHUNK)
            pltpu.make_async_copy(table.at[gbufs[0].at[pl.ds(0, C)]],
                                  rb, sg).wait()
            sb_ = sbufs[cur]
            for j in range(C // 16):
                tc[pl.ds(j * 16, 16)] = sb_[pl.ds(cl * C + j * 16, 16)]
            pltpu.sync_copy(rb, acc.at[tc], add=True)
            k2 = k + 2
            if k2 < NCH:
                sc2, cl2 = divmod(k2, NCH_S)
                pltpu.async_copy(
                    table.at[gbufs[sc2 % 2].at[pl.ds(cl2 * C, C)]], rb, sg)

    plsc.subcore_barrier()
    rs_per = NIP // NSUB
    pltpu.sync_copy(acc.at[pl.ds(sub * rs_per, rs_per)],
                    out_p.at[core, pl.ds(sub * rs_per, rs_per)])


def _aggregate_items(table, gidx, sidx):
    scratch = [
        pltpu.VMEM_SHARED((NIP + 16, H), F32),
        pltpu.VMEM((SCHUNK,), jnp.int32),
        pltpu.VMEM((SCHUNK,), jnp.int32),
        pltpu.VMEM((SCHUNK,), jnp.int32),
        pltpu.VMEM((SCHUNK,), jnp.int32),
        pltpu.VMEM((C,), jnp.int32),
        pltpu.VMEM((C,), jnp.int32),
        pltpu.VMEM((C, H), F32),
        pltpu.VMEM((C, H), F32),
        pltpu.SemaphoreType.DMA,
        pltpu.SemaphoreType.DMA,
        pltpu.SemaphoreType.DMA,
        pltpu.SemaphoreType.DMA,
    ]
    fn = pl.kernel(
        _agg_items_body,
        out_type=jax.ShapeDtypeStruct((NCORES, NIP, H), F32),
        mesh=_mesh,
        scratch_types=scratch,
        compiler_params=_sc_params,
    )
    return fn(table, gidx, sidx)


def _count_body(src, dst, out_cs, out_ci, cs, ci, gb, sb, sem):
    core = lax.axis_index("c")
    sub = lax.axis_index("s")
    ebase = (core * NSUB + sub) * EPW
    zeros16 = jnp.zeros((16,), F32)
    ones16 = jnp.full((16,), 1.0, F32)

    @pl.loop(0, NSP // 16)
    def _(q):
        cs[pl.ds(q * 16, 16)] = zeros16

    @pl.loop(0, NIP // 16)
    def _(q):
        ci[pl.ds(q * 16, 16)] = zeros16

    for sc in range(EPW // SCHUNK):
        pltpu.sync_copy(src.at[pl.ds(ebase + sc * SCHUNK, SCHUNK)], sb)
        pltpu.sync_copy(dst.at[pl.ds(ebase + sc * SCHUNK, SCHUNK)], gb)

        @pl.loop(0, SCHUNK // 64)
        def _(q):
            for jj in range(4):
                off = q * 64 + jj * 16
                s16 = sb[pl.ds(off, 16)]
                d16 = gb[pl.ds(off, 16)]
                plsc.addupdate_scatter(cs, [s16], ones16)
                plsc.addupdate_scatter(ci, [d16], ones16)

    pltpu.sync_copy(cs, out_cs.at[core, sub])
    pltpu.sync_copy(ci, out_ci.at[core, sub])


def _counts(edge_src, edge_dst):
    fn = pl.kernel(
        _count_body,
        out_type=(jax.ShapeDtypeStruct((NCORES, NSUB, NSP), F32),
                  jax.ShapeDtypeStruct((NCORES, NSUB, NIP), F32)),
        mesh=_mesh,
        scratch_types=[
            pltpu.VMEM((NSP,), F32),
            pltpu.VMEM((NIP,), F32),
            pltpu.VMEM((SCHUNK,), jnp.int32),
            pltpu.VMEM((SCHUNK,), jnp.int32),
            pltpu.SemaphoreType.DMA,
        ],
        compiler_params=_sc_params,
    )
    return fn(edge_src, edge_dst)


FC = 2000


def _final_body(a_hbm, b_hbm, c_hbm, src_hbm, dst_hbm, out_hbm,
                a_v, b_v, sbuf, dbuf, cbuf, obuf, sem):
    core = lax.axis_index("c")
    sub = lax.axis_index("s")
    wid = core * NSUB + sub
    base = wid * EPW
    pltpu.sync_copy(a_hbm, a_v)
    pltpu.sync_copy(b_hbm, b_v)
    for ch in range(EPW // FC):
        off = base + ch * FC
        pltpu.sync_copy(src_hbm.at[pl.ds(off, FC)], sbuf)
        pltpu.sync_copy(dst_hbm.at[pl.ds(off, FC)], dbuf)
        pltpu.sync_copy(c_hbm.at[pl.ds(off, FC)], cbuf)

        @pl.loop(0, FC // 16)
        def _(j):
            s16 = sbuf[pl.ds(j * 16, 16)]
            d16 = dbuf[pl.ds(j * 16, 16)]
            av = plsc.load_gather(a_v, [s16])
            bv = plsc.load_gather(b_v, [d16])
            cv = cbuf[pl.ds(j * 16, 16)]
            obuf[pl.ds(j * 16, 16)] = av + bv + cv

        pltpu.sync_copy(obuf, out_hbm.at[pl.ds(off, FC)])


def _final(a_s, b_i, c_e, src, dst):
    fn = pl.kernel(
        _final_body,
        out_type=jax.ShapeDtypeStruct((E,), F32),
        mesh=_mesh,
        scratch_types=[
            pltpu.VMEM((NSP,), F32),
            pltpu.VMEM((NIP,), F32),
            pltpu.VMEM((FC,), jnp.int32),
            pltpu.VMEM((FC,), jnp.int32),
            pltpu.VMEM((FC,), F32),
            pltpu.VMEM((FC,), F32),
            pltpu.SemaphoreType.DMA,
        ],
        compiler_params=_sc_params,
    )
    return fn(a_s, b_i, c_e, src, dst)


BR = 512


def _embed_tc_body(x_ref, w_ref, b_ref, e_ref, o_ref):
    o_ref[...] = (jnp.dot(x_ref[...], w_ref[...], preferred_element_type=F32,
                          precision=HIGH) + b_ref[...] + e_ref[...])


def _embed_tc(x, w, b, emb):
    n = x.shape[0]
    return pl.pallas_call(
        _embed_tc_body,
        grid=(n // BR,),
        in_specs=[pl.BlockSpec((BR, D), lambda i: (i, 0)),
                  pl.BlockSpec((D, H), lambda i: (0, 0)),
                  pl.BlockSpec((1, H), lambda i: (0, 0)),
                  pl.BlockSpec((BR, H), lambda i: (i, 0))],
        out_specs=pl.BlockSpec((BR, H), lambda i: (i, 0)),
        out_shape=jax.ShapeDtypeStruct((n, H), F32),
    )(x, w, b.reshape(1, H), emb)


def _sage_tc_body(relu, proj, xd_ref, p0_ref, p1_ref, cnt_ref, wr_ref, wn_ref,
                  b_ref, *rest):
    if proj:
        wp_ref, cb_ref, o_ref = rest
    else:
        (o_ref,) = rest
    cnt = jnp.sum(cnt_ref[...], axis=0)
    recip = 1.0 / jnp.maximum(cnt, 1.0)
    mean = (p0_ref[0] + p1_ref[0]) * recip[:, None]
    res = (jnp.dot(xd_ref[...], wr_ref[...], preferred_element_type=F32,
                   precision=HIGH)
           + jnp.dot(mean, wn_ref[...], preferred_element_type=F32,
                     precision=HIGH)
           + b_ref[...])
    if relu:
        res = jnp.maximum(res, 0.0)
    if proj:
        res = (jnp.dot(res, wp_ref[...], preferred_element_type=F32,
                       precision=HIGH) + cb_ref[...])
    o_ref[...] = res


def _sage_tc(xd, parts, colblk, cnt, wr, wn, b, relu=False, proj=None,
             proj_b=None):
    n = xd.shape[0]
    ins = [xd, parts, parts, cnt, wr, wn, b.reshape(1, H)]
    in_specs = [pl.BlockSpec((BR, H), lambda i: (i, 0)),
                pl.BlockSpec((1, BR, H), lambda i, cb=colblk: (0, i, cb)),
                pl.BlockSpec((1, BR, H), lambda i, cb=colblk: (1, i, cb)),
                pl.BlockSpec((NW, BR), lambda i: (0, i)),
                pl.BlockSpec((H, H), lambda i: (0, 0)),
                pl.BlockSpec((H, H), lambda i: (0, 0)),
                pl.BlockSpec((1, H), lambda i: (0, 0))]
    if proj is not None:
        ins += [proj, proj_b]
        in_specs += [pl.BlockSpec((H, H), lambda i: (0, 0)),
                     pl.BlockSpec((1, H), lambda i: (0, 0))]
    return pl.pallas_call(
        functools.partial(_sage_tc_body, relu, proj is not None),
        grid=(n // BR,),
        in_specs=in_specs,
        out_specs=pl.BlockSpec((BR, H), lambda i: (i, 0)),
        out_shape=jax.ShapeDtypeStruct((n, H), F32),
    )(*ins)


def _edgec_tc_body(ea_ref, w_ref, o_ref):
    o_ref[...] = jnp.dot(ea_ref[...], w_ref[...], preferred_element_type=F32,
                         precision=HIGH)


def _edgec_tc(ea, w):
    blk = 4096
    return pl.pallas_call(
        _edgec_tc_body,
        grid=(E // blk,),
        in_specs=[pl.BlockSpec((blk, DE), lambda i: (i, 0)),
                  pl.BlockSpec((DE, 8), lambda i: (0, 0))],
        out_specs=pl.BlockSpec((blk, 8), lambda i: (i, 0)),
        out_shape=jax.ShapeDtypeStruct((E, 8), F32),
    )(ea, w)


def kernel(student_x, item_x, student_node_id, item_node_id, edge_src,
           edge_dst, edge_attr, slW, slb, ilW, ilb, s_emb, i_emb,
           c1_si_r, c1_si_n, c1_si_b, c1_is_r, c1_is_n, c1_is_b,
           c2_si_r, c2_si_n, c2_si_b, c2_is_r, c2_is_n, c2_is_b,
           clsW, clsb):
    sx = jnp.pad(student_x, ((0, NSP - N_S), (0, 0)))
    se = jnp.pad(s_emb, ((0, NSP - N_S), (0, 0)))
    ix = jnp.pad(item_x, ((0, NIP - N_I), (0, 0)))
    ie = jnp.pad(i_emb, ((0, NIP - N_I), (0, 0)))

    s = _embed_tc(sx, slW, slb, se)
    i = _embed_tc(ix, ilW, ilb, ie)

    cs, ci = _counts(edge_src, edge_dst)
    cnt_i = ci.reshape(NW, NIP)
    cnt_s = cs.reshape(NW, NSP)
    pi = _aggregate_items(s, edge_src, edge_dst)
    i1 = _sage_tc(i, pi, 0, cnt_i, c1_si_r, c1_si_n, c1_si_b, relu=True)

    ps, ps2 = _aggregate2(i, i1, edge_dst, edge_src, NSP, NSP // 8)
    s1 = _sage_tc(s, ps, 0, cnt_s, c1_is_r, c1_is_n, c1_is_b, relu=True)
    pi2 = _aggregate_items(s1, edge_src, edge_dst)

    w_s = jnp.pad(clsW[:H], ((0, 0), (0, H - 1)))
    w_i = jnp.pad(clsW[H:2 * H], ((0, 0), (0, H - 1)))
    cb128 = jnp.pad(clsb.reshape(1, 1), ((0, 0), (0, H - 1)))
    zb128 = jnp.zeros((1, H), F32)

    a_s = _sage_tc(s1, ps2, 0, cnt_s, c2_is_r, c2_is_n, c2_is_b,
                   proj=w_s, proj_b=cb128)[:, 0]
    b_i = _sage_tc(i1, pi2, 0, cnt_i, c2_si_r, c2_si_n, c2_si_b,
                   proj=w_i, proj_b=zb128)[:, 0]

    w_e = jnp.pad(clsW[2 * H:], ((0, 0), (0, 7)))
    c_e = _edgec_tc(edge_attr, w_e)[:, 0]

    return _final(a_s, b_i, c_e, edge_src, edge_dst)
